# baseline (speedup 1.0000x reference)
import functools

import jax
import jax.numpy as jnp
from jax import lax
from jax.experimental import pallas as pl
from jax.experimental.pallas import tpu as pltpu
from jax.experimental.pallas import tpu_sc as plsc

HID = 64
NG = 64
HALF = HID // 2


@functools.cache
def _make_edge_segsum(N, n_slabs):
    NC, NS, CH = 2, 16, 128
    NB = 6
    SLK = 18
    slabs_per_tile = n_slabs // NS
    groups_per_slab = SLK // NB
    ZR = 128
    nz_full = N // ZR
    ztail = N - nz_full * ZR
    nz_iter = (nz_full + NS - 1) // NS

    mesh = plsc.VectorSubcoreMesh(core_axis_name="c", subcore_axis_name="s")

    @functools.partial(
        pl.kernel,
        out_type=jax.ShapeDtypeStruct((NC * N, HALF), jnp.float32),
        mesh=mesh,
        scratch_types=[
            pltpu.VMEM((SLK, CH), jnp.int32),
            pltpu.VMEM((SLK, CH), jnp.int32),
            pltpu.VMEM((NB, CH), jnp.int32),
            pltpu.VMEM((NB, CH, HALF), jnp.float32),
            pltpu.VMEM_SHARED((N + 8, HALF), jnp.float32),
        ] + [pltpu.SemaphoreType.DMA] * (2 * NB),
        compiler_params=pltpu.CompilerParams(use_tc_tiling_on_sc=False),
    )
    def seg(h2, src2d, dst2d, out, srcsl, dstsl, idxb, rows, acc, *sems):
        gsems = sems[:NB]
        ssems = sems[NB:]
        c = lax.axis_index("c")
        s = lax.axis_index("s")
        zero16 = jnp.zeros((16,), jnp.float32)

        def zb(i, carry):
            for b in range(NB):
                rows[b, i, pl.ds(0, 16)] = zero16
                rows[b, i, pl.ds(16, 16)] = zero16
            return carry

        lax.fori_loop(0, CH, zb, 0)

        def zr(k, carry):
            zc = s + k * NS

            @pl.when(zc < nz_full)
            def _():
                pltpu.sync_copy(rows.at[0], acc.at[pl.ds(zc * ZR, ZR)])

            return carry

        lax.fori_loop(0, nz_iter, zr, 0)
        if ztail:
            @pl.when(s == 0)
            def _():
                pltpu.sync_copy(rows.at[0].at[pl.ds(0, ztail)],
                                acc.at[pl.ds(nz_full * ZR, ztail)])
        plsc.subcore_barrier()

        def slab_body(sl_i, carry):
            slab = s * slabs_per_tile + sl_i
            base_chunk = slab * SLK
            pltpu.sync_copy(src2d.at[pl.ds(base_chunk, SLK)], srcsl)
            pltpu.sync_copy(dst2d.at[pl.ds(base_chunk, SLK)], dstsl)

            def group(gi, gcarry):
                base = gi * NB
                for b in range(NB):
                    j = base + b

                    @pl.when(gi > 0)
                    def _():
                        pltpu.make_async_copy(
                            rows.at[b], acc.at[dstsl.at[0]], ssems[b]
                        ).wait()

                    for i in range(CH // 16):
                        slc = pl.ds(i * 16, 16)
                        idxb[b, slc] = srcsl[j, slc] * 2 + c
                    pltpu.async_copy(h2.at[idxb.at[b]], rows.at[b], gsems[b])
                for b in range(NB):
                    j = base + b
                    pltpu.make_async_copy(
                        h2.at[idxb.at[b]], rows.at[b], gsems[b]
                    ).wait()
                    pltpu.async_copy(
                        rows.at[b], acc.at[dstsl.at[j]], ssems[b], add=True
                    )
                return gcarry

            lax.fori_loop(0, groups_per_slab, group, 0)
            for b in range(NB):
                pltpu.make_async_copy(
                    rows.at[b], acc.at[dstsl.at[0]], ssems[b]
                ).wait()
            return carry

        lax.fori_loop(0, slabs_per_tile, slab_body, 0)
        plsc.subcore_barrier()

        def wb(k, carry):
            zc = s + k * NS

            @pl.when(zc < nz_full)
            def _():
                pltpu.sync_copy(acc.at[pl.ds(zc * ZR, ZR)],
                                out.at[pl.ds(c * N + zc * ZR, ZR)])

            return carry

        lax.fori_loop(0, nz_iter, wb, 0)
        if ztail:
            @pl.when(s == 0)
            def _():
                pltpu.sync_copy(acc.at[pl.ds(nz_full * ZR, ztail)],
                                out.at[pl.ds(c * N + nz_full * ZR, ztail)])

    return seg


def _pool_contrib(batch_ref, hn, bn):
    bt = batch_ref[0]
    ohT = (lax.broadcasted_iota(jnp.int32, (NG, bn), 0) == bt).astype(jnp.float32)
    return jnp.dot(ohT, hn, preferred_element_type=jnp.float32)


@functools.cache
def _make_pre(N, F, bn):
    G = N // bn

    def body(x_ref, batch_ref, w_ref, b_ref, h_ref, pool_ref):
        i = pl.program_id(0)
        h = jnp.dot(x_ref[...], w_ref[...], preferred_element_type=jnp.float32)
        h = h + b_ref[...]
        h_ref[...] = h
        contrib = _pool_contrib(batch_ref, h, bn)

        @pl.when(i == 0)
        def _():
            pool_ref[...] = contrib

        @pl.when(i != 0)
        def _():
            pool_ref[...] = pool_ref[...] + contrib

    return pl.pallas_call(
        body,
        grid=(G,),
        in_specs=[
            pl.BlockSpec((bn, F), lambda i: (i, 0)),
            pl.BlockSpec((1, 1, bn), lambda i: (i, 0, 0)),
            pl.BlockSpec((F, HID), lambda i: (0, 0)),
            pl.BlockSpec((1, HID), lambda i: (0, 0)),
        ],
        out_specs=[
            pl.BlockSpec((bn, HID), lambda i: (i, 0)),
            pl.BlockSpec((NG, HID), lambda i: (0, 0)),
        ],
        out_shape=[
            jax.ShapeDtypeStruct((N, HID), jnp.float32),
            jax.ShapeDtypeStruct((NG, HID), jnp.float32),
        ],
    )


@functools.cache
def _make_mlp(N, bn, residual):
    G = N // bn

    def body(*refs):
        if residual:
            (h_ref, agg_ref, hres_ref, batch_ref, w1_ref, b1_ref, w2_ref,
             b2_ref, hout_ref, hresout_ref, pool_ref) = refs
        else:
            (h_ref, agg_ref, batch_ref, w1_ref, b1_ref, w2_ref, b2_ref,
             hout_ref, pool_ref) = refs
        i = pl.program_id(0)
        a = jnp.concatenate([agg_ref[0], agg_ref[1]], axis=1)
        z = h_ref[...] + a
        z = jnp.dot(z, w1_ref[...], preferred_element_type=jnp.float32) + b1_ref[...]
        z = jnp.maximum(z, 0.0)
        z = jnp.dot(z, w2_ref[...], preferred_element_type=jnp.float32) + b2_ref[...]
        if residual:
            z = z + hres_ref[...]
            hresout_ref[...] = z
        hn = jnp.maximum(z, 0.0)
        hout_ref[...] = hn
        contrib = _pool_contrib(batch_ref, hn, bn)

        @pl.when(i == 0)
        def _():
            pool_ref[...] = contrib

        @pl.when(i != 0)
        def _():
            pool_ref[...] = pool_ref[...] + contrib

    in_specs = [
        pl.BlockSpec((bn, HID), lambda i: (i, 0)),
        pl.BlockSpec((2, bn, HALF), lambda i: (0, i, 0)),
    ]
    if residual:
        in_specs.append(pl.BlockSpec((bn, HID), lambda i: (i, 0)))
    in_specs += [
        pl.BlockSpec((1, 1, bn), lambda i: (i, 0, 0)),
        pl.BlockSpec((HID, HID), lambda i: (0, 0)),
        pl.BlockSpec((1, HID), lambda i: (0, 0)),
        pl.BlockSpec((HID, HID), lambda i: (0, 0)),
        pl.BlockSpec((1, HID), lambda i: (0, 0)),
    ]
    out_specs = [pl.BlockSpec((bn, HID), lambda i: (i, 0))]
    out_shape = [jax.ShapeDtypeStruct((N, HID), jnp.float32)]
    if residual:
        out_specs.append(pl.BlockSpec((bn, HID), lambda i: (i, 0)))
        out_shape.append(jax.ShapeDtypeStruct((N, HID), jnp.float32))
    out_specs.append(pl.BlockSpec((NG, HID), lambda i: (0, 0)))
    out_shape.append(jax.ShapeDtypeStruct((NG, HID), jnp.float32))

    return pl.pallas_call(
        body,
        grid=(G,),
        in_specs=in_specs,
        out_specs=out_specs,
        out_shape=out_shape,
    )


@functools.cache
def _make_post(OUT, CAT):
    def body(p_ref, w1_ref, b1_ref, w2_ref, b2_ref, out_ref):
        p = p_ref[...]
        z = jnp.dot(p, w1_ref[...], preferred_element_type=jnp.float32) + b1_ref[...]
        z = jnp.maximum(z, 0.0)
        out_ref[...] = (
            jnp.dot(z, w2_ref[...], preferred_element_type=jnp.float32) + b2_ref[...]
        )

    return pl.pallas_call(
        body,
        out_shape=jax.ShapeDtypeStruct((NG, OUT), jnp.float32),
    )


def kernel(x, edge_index, batch, params):
    N, F = x.shape
    E = edge_index.shape[1]
    OUT = params["post_W2"].shape[1]
    src = edge_index[0]
    dst = edge_index[1]
    bn = 2000
    G = N // bn
    batch3 = batch.reshape(G, 1, bn)

    SLAB_E = 16 * 18 * 128
    E_pad = -(-E // SLAB_E) * SLAB_E
    n_slabs = E_pad // (18 * 128)
    if E_pad != E:
        src = jnp.concatenate([src, jnp.zeros((E_pad - E,), jnp.int32)])
        dst = jnp.concatenate([dst, jnp.full((E_pad - E,), N, jnp.int32)])
    src2d = src.reshape(E_pad // 128, 128)
    dst2d = dst.reshape(E_pad // 128, 128)

    pre = _make_pre(N, F, bn)
    h, p0 = pre(x, batch3, params["pre_W"], params["pre_b"].reshape(1, HID))

    segsum = _make_edge_segsum(N, n_slabs)
    mlp_plain = _make_mlp(N, bn, False)
    mlp_res = _make_mlp(N, bn, True)

    pools = [p0]
    hres = h
    for l in range(4):
        agg2 = segsum(h.reshape(2 * N, HALF), src2d, dst2d).reshape(2, N, HALF)
        w1 = params["conv%d_W1" % l]
        b1 = params["conv%d_b1" % l].reshape(1, HID)
        w2 = params["conv%d_W2" % l]
        b2 = params["conv%d_b2" % l].reshape(1, HID)
        if l & 1:
            h, hres, pc = mlp_res(h, agg2, hres, batch3, w1, b1, w2, b2)
        else:
            h, pc = mlp_plain(h, agg2, batch3, w1, b1, w2, b2)
        pools.append(pc)

    pooled = jnp.concatenate(pools, axis=1)
    post = _make_post(OUT, pooled.shape[1])
    return post(
        pooled,
        params["post_W1"],
        params["post_b1"].reshape(1, HID),
        params["post_W2"],
        params["post_b2"].reshape(1, OUT),
    )


# speedup vs baseline: 1.5902x; 1.5902x over previous
import functools

import jax
import jax.numpy as jnp
from jax import lax
from jax.experimental import pallas as pl
from jax.experimental.pallas import tpu as pltpu
from jax.experimental.pallas import tpu_sc as plsc

HID = 64
NG = 64
HALF = HID // 2


@functools.cache
def _make_edge_segsum(N, E):
    NC, NS, CH = 2, 16, 128
    NB = 4
    SLK = 32
    nchunk = E // CH
    full_slabs = (nchunk // SLK) // NS * NS
    slabs_per_tile = full_slabs // NS
    groups_per_slab = SLK // NB
    tail0 = full_slabs * SLK
    tail_chunks = nchunk - tail0
    tail_iter = (tail_chunks + NS - 1) // NS
    ZR = 128
    nz_full = N // ZR
    ztail = N - nz_full * ZR
    nz_iter = (nz_full + NS - 1) // NS

    mesh = plsc.VectorSubcoreMesh(core_axis_name="c", subcore_axis_name="s")

    @functools.partial(
        pl.kernel,
        out_type=jax.ShapeDtypeStruct((NC * N, HALF), jnp.float32),
        mesh=mesh,
        scratch_types=[
            pltpu.VMEM((SLK, CH), jnp.int32),
            pltpu.VMEM((SLK, CH), jnp.int32),
            pltpu.VMEM((NB, CH), jnp.int32),
            pltpu.VMEM((NB, CH, HALF), jnp.float32),
            pltpu.VMEM_SHARED((N, HALF), jnp.float32),
        ] + [pltpu.SemaphoreType.DMA] * (2 * NB),
        compiler_params=pltpu.CompilerParams(use_tc_tiling_on_sc=False),
    )
    def seg(h2, src2d, dst2d, out, srcsl, dstsl, idxb, rows, acc, *sems):
        gsems = sems[:NB]
        ssems = sems[NB:]
        c = lax.axis_index("c")
        s = lax.axis_index("s")
        zero16 = jnp.zeros((16,), jnp.float32)

        def zb(i, carry):
            for b in range(NB):
                rows[b, i, pl.ds(0, 16)] = zero16
                rows[b, i, pl.ds(16, 16)] = zero16
            return carry

        lax.fori_loop(0, CH, zb, 0)

        def zr(k, carry):
            zc = s + k * NS

            @pl.when(zc < nz_full)
            def _():
                pltpu.sync_copy(rows.at[0], acc.at[pl.ds(zc * ZR, ZR)])

            return carry

        lax.fori_loop(0, nz_iter, zr, 0)
        if ztail:
            @pl.when(s == 0)
            def _():
                pltpu.sync_copy(rows.at[0].at[pl.ds(0, ztail)],
                                acc.at[pl.ds(nz_full * ZR, ztail)])
        plsc.subcore_barrier()

        def slab_body(sl_i, carry):
            slab = s * slabs_per_tile + sl_i
            base_chunk = slab * SLK
            pltpu.sync_copy(src2d.at[pl.ds(base_chunk, SLK)], srcsl)
            pltpu.sync_copy(dst2d.at[pl.ds(base_chunk, SLK)], dstsl)

            def group(gi, gcarry):
                base = gi * NB
                for b in range(NB):
                    j = base + b

                    @pl.when(gi > 0)
                    def _():
                        pltpu.make_async_copy(
                            rows.at[b], acc.at[dstsl.at[0]], ssems[b]
                        ).wait()

                    for i in range(CH // 16):
                        slc = pl.ds(i * 16, 16)
                        idxb[b, slc] = srcsl[j, slc] * 2 + c
                    pltpu.async_copy(h2.at[idxb.at[b]], rows.at[b], gsems[b])
                for b in range(NB):
                    j = base + b
                    pltpu.make_async_copy(
                        h2.at[idxb.at[b]], rows.at[b], gsems[b]
                    ).wait()
                    pltpu.async_copy(
                        rows.at[b], acc.at[dstsl.at[j]], ssems[b], add=True
                    )
                return gcarry

            lax.fori_loop(0, groups_per_slab, group, 0)
            for b in range(NB):
                pltpu.make_async_copy(
                    rows.at[b], acc.at[dstsl.at[0]], ssems[b]
                ).wait()
            return carry

        lax.fori_loop(0, slabs_per_tile, slab_body, 0)

        if tail_chunks:
            def tail(k, carry):
                t = tail0 + s + k * NS

                @pl.when(t < nchunk)
                def _():
                    pltpu.sync_copy(src2d.at[t], srcsl.at[0])
                    pltpu.sync_copy(dst2d.at[t], dstsl.at[0])
                    for i in range(CH // 16):
                        slc = pl.ds(i * 16, 16)
                        idxb[0, slc] = srcsl[0, slc] * 2 + c
                    pltpu.async_copy(
                        h2.at[idxb.at[0]], rows.at[0], gsems[0]
                    ).wait()
                    pltpu.sync_copy(rows.at[0], acc.at[dstsl.at[0]], add=True)

                return carry

            lax.fori_loop(0, tail_iter, tail, 0)

        plsc.subcore_barrier()

        def wb(k, carry):
            zc = s + k * NS

            @pl.when(zc < nz_full)
            def _():
                pltpu.sync_copy(acc.at[pl.ds(zc * ZR, ZR)],
                                out.at[pl.ds(c * N + zc * ZR, ZR)])

            return carry

        lax.fori_loop(0, nz_iter, wb, 0)
        if ztail:
            @pl.when(s == 0)
            def _():
                pltpu.sync_copy(acc.at[pl.ds(nz_full * ZR, ztail)],
                                out.at[pl.ds(c * N + nz_full * ZR, ztail)])

    return seg


def _pool_contrib(batch_ref, hn, bn):
    bt = batch_ref[0]
    ohT = (lax.broadcasted_iota(jnp.int32, (NG, bn), 0) == bt).astype(jnp.float32)
    return jnp.dot(ohT, hn, preferred_element_type=jnp.float32)


@functools.cache
def _make_pre(N, F, bn):
    G = N // bn

    def body(x_ref, batch_ref, w_ref, b_ref, h_ref, pool_ref):
        i = pl.program_id(0)
        h = jnp.dot(x_ref[...], w_ref[...], preferred_element_type=jnp.float32)
        h = h + b_ref[...]
        h_ref[...] = h
        contrib = _pool_contrib(batch_ref, h, bn)

        @pl.when(i == 0)
        def _():
            pool_ref[...] = contrib

        @pl.when(i != 0)
        def _():
            pool_ref[...] = pool_ref[...] + contrib

    return pl.pallas_call(
        body,
        grid=(G,),
        in_specs=[
            pl.BlockSpec((bn, F), lambda i: (i, 0)),
            pl.BlockSpec((1, 1, bn), lambda i: (i, 0, 0)),
            pl.BlockSpec((F, HID), lambda i: (0, 0)),
            pl.BlockSpec((1, HID), lambda i: (0, 0)),
        ],
        out_specs=[
            pl.BlockSpec((bn, HID), lambda i: (i, 0)),
            pl.BlockSpec((NG, HID), lambda i: (0, 0)),
        ],
        out_shape=[
            jax.ShapeDtypeStruct((N, HID), jnp.float32),
            jax.ShapeDtypeStruct((NG, HID), jnp.float32),
        ],
    )


@functools.cache
def _make_mlp(N, bn, residual):
    G = N // bn

    def body(*refs):
        if residual:
            (h_ref, agg_ref, hres_ref, batch_ref, w1_ref, b1_ref, w2_ref,
             b2_ref, hout_ref, hresout_ref, pool_ref) = refs
        else:
            (h_ref, agg_ref, batch_ref, w1_ref, b1_ref, w2_ref, b2_ref,
             hout_ref, pool_ref) = refs
        i = pl.program_id(0)
        a = jnp.concatenate([agg_ref[0], agg_ref[1]], axis=1)
        z = h_ref[...] + a
        z = jnp.dot(z, w1_ref[...], preferred_element_type=jnp.float32) + b1_ref[...]
        z = jnp.maximum(z, 0.0)
        z = jnp.dot(z, w2_ref[...], preferred_element_type=jnp.float32) + b2_ref[...]
        if residual:
            z = z + hres_ref[...]
            hresout_ref[...] = z
        hn = jnp.maximum(z, 0.0)
        hout_ref[...] = hn
        contrib = _pool_contrib(batch_ref, hn, bn)

        @pl.when(i == 0)
        def _():
            pool_ref[...] = contrib

        @pl.when(i != 0)
        def _():
            pool_ref[...] = pool_ref[...] + contrib

    in_specs = [
        pl.BlockSpec((bn, HID), lambda i: (i, 0)),
        pl.BlockSpec((2, bn, HALF), lambda i: (0, i, 0)),
    ]
    if residual:
        in_specs.append(pl.BlockSpec((bn, HID), lambda i: (i, 0)))
    in_specs += [
        pl.BlockSpec((1, 1, bn), lambda i: (i, 0, 0)),
        pl.BlockSpec((HID, HID), lambda i: (0, 0)),
        pl.BlockSpec((1, HID), lambda i: (0, 0)),
        pl.BlockSpec((HID, HID), lambda i: (0, 0)),
        pl.BlockSpec((1, HID), lambda i: (0, 0)),
    ]
    out_specs = [pl.BlockSpec((bn, HID), lambda i: (i, 0))]
    out_shape = [jax.ShapeDtypeStruct((N, HID), jnp.float32)]
    if residual:
        out_specs.append(pl.BlockSpec((bn, HID), lambda i: (i, 0)))
        out_shape.append(jax.ShapeDtypeStruct((N, HID), jnp.float32))
    out_specs.append(pl.BlockSpec((NG, HID), lambda i: (0, 0)))
    out_shape.append(jax.ShapeDtypeStruct((NG, HID), jnp.float32))

    return pl.pallas_call(
        body,
        grid=(G,),
        in_specs=in_specs,
        out_specs=out_specs,
        out_shape=out_shape,
    )


@functools.cache
def _make_post(OUT, CAT):
    def body(p_ref, w1_ref, b1_ref, w2_ref, b2_ref, out_ref):
        p = p_ref[...]
        z = jnp.dot(p, w1_ref[...], preferred_element_type=jnp.float32) + b1_ref[...]
        z = jnp.maximum(z, 0.0)
        out_ref[...] = (
            jnp.dot(z, w2_ref[...], preferred_element_type=jnp.float32) + b2_ref[...]
        )

    return pl.pallas_call(
        body,
        out_shape=jax.ShapeDtypeStruct((NG, OUT), jnp.float32),
    )


def kernel(x, edge_index, batch, params):
    N, F = x.shape
    E = edge_index.shape[1]
    OUT = params["post_W2"].shape[1]
    src = edge_index[0]
    dst = edge_index[1]
    bn = 2000
    G = N // bn
    batch3 = batch.reshape(G, 1, bn)

    src2d = src.reshape(E // 128, 128)
    dst2d = dst.reshape(E // 128, 128)

    pre = _make_pre(N, F, bn)
    h, p0 = pre(x, batch3, params["pre_W"], params["pre_b"].reshape(1, HID))

    segsum = _make_edge_segsum(N, E)
    mlp_plain = _make_mlp(N, bn, False)
    mlp_res = _make_mlp(N, bn, True)

    pools = [p0]
    hres = h
    for l in range(4):
        agg2 = segsum(h.reshape(2 * N, HALF), src2d, dst2d).reshape(2, N, HALF)
        w1 = params["conv%d_W1" % l]
        b1 = params["conv%d_b1" % l].reshape(1, HID)
        w2 = params["conv%d_W2" % l]
        b2 = params["conv%d_b2" % l].reshape(1, HID)
        if l & 1:
            h, hres, pc = mlp_res(h, agg2, hres, batch3, w1, b1, w2, b2)
        else:
            h, pc = mlp_plain(h, agg2, batch3, w1, b1, w2, b2)
        pools.append(pc)

    pooled = jnp.concatenate(pools, axis=1)
    post = _make_post(OUT, pooled.shape[1])
    return post(
        pooled,
        params["post_W1"],
        params["post_b1"].reshape(1, HID),
        params["post_W2"],
        params["post_b2"].reshape(1, OUT),
    )


# speedup vs baseline: 1.5919x; 1.0011x over previous
import functools

import jax
import jax.numpy as jnp
from jax import lax
from jax.experimental import pallas as pl
from jax.experimental.pallas import tpu as pltpu
from jax.experimental.pallas import tpu_sc as plsc

HID = 64
NG = 64
HALF = HID // 2


@functools.cache
def _make_edge_segsum(N, E):
    NC, NS, CH = 2, 16, 128
    NB = 4
    SLK = 32
    nchunk = E // CH
    full_slabs = (nchunk // SLK) // NS * NS
    slabs_per_tile = full_slabs // NS
    groups_per_slab = SLK // NB
    tail0 = full_slabs * SLK
    tail_chunks = nchunk - tail0
    tail_iter = (tail_chunks + NS - 1) // NS
    ZR = 128
    nz_full = N // ZR
    ztail = N - nz_full * ZR
    nz_iter = (nz_full + NS - 1) // NS

    mesh = plsc.VectorSubcoreMesh(core_axis_name="c", subcore_axis_name="s")

    @functools.partial(
        pl.kernel,
        out_type=jax.ShapeDtypeStruct((NC * N, HALF), jnp.float32),
        mesh=mesh,
        scratch_types=[
            pltpu.VMEM((SLK, CH), jnp.int32),
            pltpu.VMEM((SLK, CH), jnp.int32),
            pltpu.VMEM((NB, CH), jnp.int32),
            pltpu.VMEM((NB, CH, HALF), jnp.float32),
            pltpu.VMEM_SHARED((N, HALF), jnp.float32),
        ] + [pltpu.SemaphoreType.DMA] * (2 * NB),
        compiler_params=pltpu.CompilerParams(use_tc_tiling_on_sc=False),
    )
    def seg(h2, src2d, dst2d, out, srcsl, dstsl, idxb, rows, acc, *sems):
        gsems = sems[:NB]
        ssems = sems[NB:]
        c = lax.axis_index("c")
        s = lax.axis_index("s")
        zero16 = jnp.zeros((16,), jnp.float32)

        def zb(i, carry):
            for b in range(NB):
                rows[b, i, pl.ds(0, 16)] = zero16
                rows[b, i, pl.ds(16, 16)] = zero16
            return carry

        lax.fori_loop(0, CH, zb, 0)

        def zr(k, carry):
            zc = s + k * NS

            @pl.when(zc < nz_full)
            def _():
                pltpu.sync_copy(rows.at[0], acc.at[pl.ds(zc * ZR, ZR)])

            return carry

        lax.fori_loop(0, nz_iter, zr, 0)
        if ztail:
            @pl.when(s == 0)
            def _():
                pltpu.sync_copy(rows.at[0].at[pl.ds(0, ztail)],
                                acc.at[pl.ds(nz_full * ZR, ztail)])
        plsc.subcore_barrier()

        def slab_body(sl_i, carry):
            slab = s * slabs_per_tile + sl_i
            base_chunk = slab * SLK
            pltpu.sync_copy(src2d.at[pl.ds(base_chunk, SLK)], srcsl)
            pltpu.sync_copy(dst2d.at[pl.ds(base_chunk, SLK)], dstsl)

            def group(gi, gcarry):
                base = gi * NB
                for b in range(NB):
                    j = base + b

                    @pl.when(gi > 0)
                    def _():
                        pltpu.make_async_copy(
                            rows.at[b], acc.at[dstsl.at[0]], ssems[b]
                        ).wait()

                    for i in range(CH // 16):
                        slc = pl.ds(i * 16, 16)
                        idxb[b, slc] = srcsl[j, slc] * 2 + c
                    pltpu.async_copy(h2.at[idxb.at[b]], rows.at[b], gsems[b])
                for b in range(NB):
                    j = base + b
                    pltpu.make_async_copy(
                        h2.at[idxb.at[b]], rows.at[b], gsems[b]
                    ).wait()
                    pltpu.async_copy(
                        rows.at[b], acc.at[dstsl.at[j]], ssems[b], add=True
                    )
                return gcarry

            lax.fori_loop(0, groups_per_slab, group, 0)
            for b in range(NB):
                pltpu.make_async_copy(
                    rows.at[b], acc.at[dstsl.at[0]], ssems[b]
                ).wait()
            return carry

        lax.fori_loop(0, slabs_per_tile, slab_body, 0)

        if tail_chunks:
            def tail(k, carry):
                t = tail0 + s + k * NS

                @pl.when(t < nchunk)
                def _():
                    pltpu.sync_copy(src2d.at[t], srcsl.at[0])
                    pltpu.sync_copy(dst2d.at[t], dstsl.at[0])
                    for i in range(CH // 16):
                        slc = pl.ds(i * 16, 16)
                        idxb[0, slc] = srcsl[0, slc] * 2 + c
                    pltpu.async_copy(
                        h2.at[idxb.at[0]], rows.at[0], gsems[0]
                    ).wait()
                    pltpu.sync_copy(rows.at[0], acc.at[dstsl.at[0]], add=True)

                return carry

            lax.fori_loop(0, tail_iter, tail, 0)

        plsc.subcore_barrier()

        def wb(k, carry):
            zc = s + k * NS

            @pl.when(zc < nz_full)
            def _():
                pltpu.sync_copy(acc.at[pl.ds(zc * ZR, ZR)],
                                out.at[pl.ds(c * N + zc * ZR, ZR)])

            return carry

        lax.fori_loop(0, nz_iter, wb, 0)
        if ztail:
            @pl.when(s == 0)
            def _():
                pltpu.sync_copy(acc.at[pl.ds(nz_full * ZR, ztail)],
                                out.at[pl.ds(c * N + nz_full * ZR, ztail)])

    return seg


def _pool_contrib(batch_ref, hn, bn):
    bt = batch_ref[0]
    ohT = (lax.broadcasted_iota(jnp.int32, (NG, bn), 0) == bt).astype(jnp.float32)
    return jnp.dot(ohT, hn, preferred_element_type=jnp.float32)


@functools.cache
def _make_pre(N, F, bn):
    G = N // bn

    def body(x_ref, batch_ref, w_ref, b_ref, h_ref, pool_ref):
        i = pl.program_id(0)
        h = jnp.dot(x_ref[...], w_ref[...], preferred_element_type=jnp.float32)
        h = h + b_ref[...]
        h_ref[...] = h
        contrib = _pool_contrib(batch_ref, h, bn)

        @pl.when(i == 0)
        def _():
            pool_ref[...] = contrib

        @pl.when(i != 0)
        def _():
            pool_ref[...] = pool_ref[...] + contrib

    return pl.pallas_call(
        body,
        grid=(G,),
        in_specs=[
            pl.BlockSpec((bn, F), lambda i: (i, 0)),
            pl.BlockSpec((1, 1, bn), lambda i: (i, 0, 0)),
            pl.BlockSpec((F, HID), lambda i: (0, 0)),
            pl.BlockSpec((1, HID), lambda i: (0, 0)),
        ],
        out_specs=[
            pl.BlockSpec((bn, HID), lambda i: (i, 0)),
            pl.BlockSpec((NG, HID), lambda i: (0, 0)),
        ],
        out_shape=[
            jax.ShapeDtypeStruct((N, HID), jnp.float32),
            jax.ShapeDtypeStruct((NG, HID), jnp.float32),
        ],
    )


@functools.cache
def _make_mlp(N, bn, residual):
    G = N // bn

    def body(*refs):
        if residual:
            (h_ref, agg_ref, hres_ref, batch_ref, w1_ref, b1_ref, w2_ref,
             b2_ref, hout_ref, hresout_ref, pool_ref) = refs
        else:
            (h_ref, agg_ref, batch_ref, w1_ref, b1_ref, w2_ref, b2_ref,
             hout_ref, pool_ref) = refs
        i = pl.program_id(0)
        a = jnp.concatenate([agg_ref[0], agg_ref[1]], axis=1)
        z = h_ref[...] + a
        z = jnp.dot(z, w1_ref[...], preferred_element_type=jnp.float32) + b1_ref[...]
        z = jnp.maximum(z, 0.0)
        z = jnp.dot(z, w2_ref[...], preferred_element_type=jnp.float32) + b2_ref[...]
        if residual:
            z = z + hres_ref[...]
            hresout_ref[...] = z
        hn = jnp.maximum(z, 0.0)
        hout_ref[...] = hn
        contrib = _pool_contrib(batch_ref, hn, bn)

        @pl.when(i == 0)
        def _():
            pool_ref[...] = contrib

        @pl.when(i != 0)
        def _():
            pool_ref[...] = pool_ref[...] + contrib

    in_specs = [
        pl.BlockSpec((bn, HID), lambda i: (i, 0)),
        pl.BlockSpec((2, bn, HALF), lambda i: (0, i, 0)),
    ]
    if residual:
        in_specs.append(pl.BlockSpec((bn, HID), lambda i: (i, 0)))
    in_specs += [
        pl.BlockSpec((1, 1, bn), lambda i: (i, 0, 0)),
        pl.BlockSpec((HID, HID), lambda i: (0, 0)),
        pl.BlockSpec((1, HID), lambda i: (0, 0)),
        pl.BlockSpec((HID, HID), lambda i: (0, 0)),
        pl.BlockSpec((1, HID), lambda i: (0, 0)),
    ]
    out_specs = [pl.BlockSpec((bn, HID), lambda i: (i, 0))]
    out_shape = [jax.ShapeDtypeStruct((N, HID), jnp.float32)]
    if residual:
        out_specs.append(pl.BlockSpec((bn, HID), lambda i: (i, 0)))
        out_shape.append(jax.ShapeDtypeStruct((N, HID), jnp.float32))
    out_specs.append(pl.BlockSpec((NG, HID), lambda i: (0, 0)))
    out_shape.append(jax.ShapeDtypeStruct((NG, HID), jnp.float32))

    return pl.pallas_call(
        body,
        grid=(G,),
        in_specs=in_specs,
        out_specs=out_specs,
        out_shape=out_shape,
        compiler_params=pltpu.CompilerParams(
            allow_input_fusion=[True] * len(in_specs)),
    )


@functools.cache
def _make_post(OUT, CAT):
    def body(p_ref, w1_ref, b1_ref, w2_ref, b2_ref, out_ref):
        p = p_ref[...]
        z = jnp.dot(p, w1_ref[...], preferred_element_type=jnp.float32) + b1_ref[...]
        z = jnp.maximum(z, 0.0)
        out_ref[...] = (
            jnp.dot(z, w2_ref[...], preferred_element_type=jnp.float32) + b2_ref[...]
        )

    return pl.pallas_call(
        body,
        out_shape=jax.ShapeDtypeStruct((NG, OUT), jnp.float32),
    )


def kernel(x, edge_index, batch, params):
    N, F = x.shape
    E = edge_index.shape[1]
    OUT = params["post_W2"].shape[1]
    src = edge_index[0]
    dst = edge_index[1]
    bn = 2000
    G = N // bn
    batch3 = batch.reshape(G, 1, bn)

    src2d = src.reshape(E // 128, 128)
    dst2d = dst.reshape(E // 128, 128)

    pre = _make_pre(N, F, bn)
    h, p0 = pre(x, batch3, params["pre_W"], params["pre_b"].reshape(1, HID))

    segsum = _make_edge_segsum(N, E)
    mlp_plain = _make_mlp(N, bn, False)
    mlp_res = _make_mlp(N, bn, True)

    pools = [p0]
    hres = h
    for l in range(4):
        agg2 = segsum(h.reshape(2 * N, HALF), src2d, dst2d).reshape(2, N, HALF)
        w1 = params["conv%d_W1" % l]
        b1 = params["conv%d_b1" % l].reshape(1, HID)
        w2 = params["conv%d_W2" % l]
        b2 = params["conv%d_b2" % l].reshape(1, HID)
        if l & 1:
            h, hres, pc = mlp_res(h, agg2, hres, batch3, w1, b1, w2, b2)
        else:
            h, pc = mlp_plain(h, agg2, batch3, w1, b1, w2, b2)
        pools.append(pc)

    pooled = jnp.concatenate(pools, axis=1)
    post = _make_post(OUT, pooled.shape[1])
    return post(
        pooled,
        params["post_W1"],
        params["post_b1"].reshape(1, HID),
        params["post_W2"],
        params["post_b2"].reshape(1, OUT),
    )


# speedup vs baseline: 1.7211x; 1.0811x over previous
import functools

import jax
import jax.numpy as jnp
from jax import lax
from jax.experimental import pallas as pl
from jax.experimental.pallas import tpu as pltpu
from jax.experimental.pallas import tpu_sc as plsc

HID = 64
NG = 64
HALF = HID // 2


@functools.cache
def _make_edge_segsum(N, E):
    NC, NS, CH = 2, 16, 128
    NB = 5
    SLK = 30
    nchunk = E // CH
    full_slabs = (nchunk // SLK) // NS * NS
    slabs_per_tile = full_slabs // NS
    groups_per_slab = SLK // NB
    tail0 = full_slabs * SLK
    tail_chunks = nchunk - tail0
    tail_iter = (tail_chunks + NS - 1) // NS
    ZR = 128
    nz_full = N // ZR
    ztail = N - nz_full * ZR
    nz_iter = (nz_full + NS - 1) // NS

    mesh = plsc.VectorSubcoreMesh(core_axis_name="c", subcore_axis_name="s")

    @functools.partial(
        pl.kernel,
        out_type=jax.ShapeDtypeStruct((NC * N, HALF), jnp.float32),
        mesh=mesh,
        scratch_types=[
            pltpu.VMEM((SLK, CH), jnp.int32),
            pltpu.VMEM((SLK, CH), jnp.int32),
            pltpu.VMEM((NB, CH), jnp.int32),
            pltpu.VMEM((NB, CH, HALF), jnp.float32),
            pltpu.VMEM_SHARED((N, HALF), jnp.float32),
        ] + [pltpu.SemaphoreType.DMA] * (2 * NB),
        compiler_params=pltpu.CompilerParams(use_tc_tiling_on_sc=False),
    )
    def seg(h2, src2d, dst2d, out, srcsl, dstsl, idxb, rows, acc, *sems):
        gsems = sems[:NB]
        ssems = sems[NB:]
        c = lax.axis_index("c")
        s = lax.axis_index("s")
        zero16 = jnp.zeros((16,), jnp.float32)

        def zb(i, carry):
            for b in range(NB):
                rows[b, i, pl.ds(0, 16)] = zero16
                rows[b, i, pl.ds(16, 16)] = zero16
            return carry

        lax.fori_loop(0, CH, zb, 0)

        def zr(k, carry):
            zc = s + k * NS

            @pl.when(zc < nz_full)
            def _():
                pltpu.sync_copy(rows.at[0], acc.at[pl.ds(zc * ZR, ZR)])

            return carry

        lax.fori_loop(0, nz_iter, zr, 0)
        if ztail:
            @pl.when(s == 0)
            def _():
                pltpu.sync_copy(rows.at[0].at[pl.ds(0, ztail)],
                                acc.at[pl.ds(nz_full * ZR, ztail)])
        plsc.subcore_barrier()

        def slab_body(sl_i, carry):
            slab = s * slabs_per_tile + sl_i
            base_chunk = slab * SLK
            pltpu.sync_copy(src2d.at[pl.ds(base_chunk, SLK)], srcsl)
            pltpu.sync_copy(dst2d.at[pl.ds(base_chunk, SLK)], dstsl)

            def group(gi, gcarry):
                base = gi * NB
                for b in range(NB):
                    j = base + b

                    @pl.when(gi > 0)
                    def _():
                        pltpu.make_async_copy(
                            rows.at[b], acc.at[dstsl.at[0]], ssems[b]
                        ).wait()

                    for i in range(CH // 16):
                        slc = pl.ds(i * 16, 16)
                        idxb[b, slc] = srcsl[j, slc] * 2 + c
                    pltpu.async_copy(h2.at[idxb.at[b]], rows.at[b], gsems[b])
                for b in range(NB):
                    j = base + b
                    pltpu.make_async_copy(
                        h2.at[idxb.at[b]], rows.at[b], gsems[b]
                    ).wait()
                    pltpu.async_copy(
                        rows.at[b], acc.at[dstsl.at[j]], ssems[b], add=True
                    )
                return gcarry

            lax.fori_loop(0, groups_per_slab, group, 0)
            for b in range(NB):
                pltpu.make_async_copy(
                    rows.at[b], acc.at[dstsl.at[0]], ssems[b]
                ).wait()
            return carry

        lax.fori_loop(0, slabs_per_tile, slab_body, 0)

        if tail_chunks:
            def tail(k, carry):
                t = tail0 + s + k * NS

                @pl.when(t < nchunk)
                def _():
                    pltpu.sync_copy(src2d.at[t], srcsl.at[0])
                    pltpu.sync_copy(dst2d.at[t], dstsl.at[0])
                    for i in range(CH // 16):
                        slc = pl.ds(i * 16, 16)
                        idxb[0, slc] = srcsl[0, slc] * 2 + c
                    pltpu.async_copy(
                        h2.at[idxb.at[0]], rows.at[0], gsems[0]
                    ).wait()
                    pltpu.sync_copy(rows.at[0], acc.at[dstsl.at[0]], add=True)

                return carry

            lax.fori_loop(0, tail_iter, tail, 0)

        plsc.subcore_barrier()

        def wb(k, carry):
            zc = s + k * NS

            @pl.when(zc < nz_full)
            def _():
                pltpu.sync_copy(acc.at[pl.ds(zc * ZR, ZR)],
                                out.at[pl.ds(c * N + zc * ZR, ZR)])

            return carry

        lax.fori_loop(0, nz_iter, wb, 0)
        if ztail:
            @pl.when(s == 0)
            def _():
                pltpu.sync_copy(acc.at[pl.ds(nz_full * ZR, ztail)],
                                out.at[pl.ds(c * N + nz_full * ZR, ztail)])

    return seg


def _pool_contrib(batch_ref, hn, bn):
    bt = batch_ref[0]
    ohT = (lax.broadcasted_iota(jnp.int32, (NG, bn), 0) == bt).astype(jnp.float32)
    return jnp.dot(ohT, hn, preferred_element_type=jnp.float32)


@functools.cache
def _make_pre(N, F, bn):
    G = N // bn

    def body(x_ref, batch_ref, w_ref, b_ref, h_ref, pool_ref):
        i = pl.program_id(0)
        h = jnp.dot(x_ref[...], w_ref[...], preferred_element_type=jnp.float32)
        h = h + b_ref[...]
        h_ref[...] = h
        contrib = _pool_contrib(batch_ref, h, bn)

        @pl.when(i == 0)
        def _():
            pool_ref[...] = contrib

        @pl.when(i != 0)
        def _():
            pool_ref[...] = pool_ref[...] + contrib

    return pl.pallas_call(
        body,
        grid=(G,),
        in_specs=[
            pl.BlockSpec((bn, F), lambda i: (i, 0)),
            pl.BlockSpec((1, 1, bn), lambda i: (i, 0, 0)),
            pl.BlockSpec((F, HID), lambda i: (0, 0)),
            pl.BlockSpec((1, HID), lambda i: (0, 0)),
        ],
        out_specs=[
            pl.BlockSpec((bn, HID), lambda i: (i, 0)),
            pl.BlockSpec((NG, HID), lambda i: (0, 0)),
        ],
        out_shape=[
            jax.ShapeDtypeStruct((N, HID), jnp.float32),
            jax.ShapeDtypeStruct((NG, HID), jnp.float32),
        ],
    )


@functools.cache
def _make_mlp(N, bn, residual):
    G = N // bn

    def body(*refs):
        if residual:
            (h_ref, agg_ref, hres_ref, batch_ref, w1_ref, b1_ref, w2_ref,
             b2_ref, hout_ref, hresout_ref, pool_ref) = refs
        else:
            (h_ref, agg_ref, batch_ref, w1_ref, b1_ref, w2_ref, b2_ref,
             hout_ref, pool_ref) = refs
        i = pl.program_id(0)
        a = jnp.concatenate([agg_ref[0], agg_ref[1]], axis=1)
        z = h_ref[...] + a
        z = jnp.dot(z, w1_ref[...], preferred_element_type=jnp.float32) + b1_ref[...]
        z = jnp.maximum(z, 0.0)
        z = jnp.dot(z, w2_ref[...], preferred_element_type=jnp.float32) + b2_ref[...]
        if residual:
            z = z + hres_ref[...]
            hresout_ref[...] = z
        hn = jnp.maximum(z, 0.0)
        hout_ref[...] = hn
        contrib = _pool_contrib(batch_ref, hn, bn)

        @pl.when(i == 0)
        def _():
            pool_ref[...] = contrib

        @pl.when(i != 0)
        def _():
            pool_ref[...] = pool_ref[...] + contrib

    in_specs = [
        pl.BlockSpec((bn, HID), lambda i: (i, 0)),
        pl.BlockSpec((2, bn, HALF), lambda i: (0, i, 0)),
    ]
    if residual:
        in_specs.append(pl.BlockSpec((bn, HID), lambda i: (i, 0)))
    in_specs += [
        pl.BlockSpec((1, 1, bn), lambda i: (i, 0, 0)),
        pl.BlockSpec((HID, HID), lambda i: (0, 0)),
        pl.BlockSpec((1, HID), lambda i: (0, 0)),
        pl.BlockSpec((HID, HID), lambda i: (0, 0)),
        pl.BlockSpec((1, HID), lambda i: (0, 0)),
    ]
    out_specs = [pl.BlockSpec((bn, HID), lambda i: (i, 0))]
    out_shape = [jax.ShapeDtypeStruct((N, HID), jnp.float32)]
    if residual:
        out_specs.append(pl.BlockSpec((bn, HID), lambda i: (i, 0)))
        out_shape.append(jax.ShapeDtypeStruct((N, HID), jnp.float32))
    out_specs.append(pl.BlockSpec((NG, HID), lambda i: (0, 0)))
    out_shape.append(jax.ShapeDtypeStruct((NG, HID), jnp.float32))

    return pl.pallas_call(
        body,
        grid=(G,),
        in_specs=in_specs,
        out_specs=out_specs,
        out_shape=out_shape,
        compiler_params=pltpu.CompilerParams(
            allow_input_fusion=[True] * len(in_specs)),
    )


@functools.cache
def _make_post(OUT, CAT):
    def body(p_ref, w1_ref, b1_ref, w2_ref, b2_ref, out_ref):
        p = p_ref[...]
        z = jnp.dot(p, w1_ref[...], preferred_element_type=jnp.float32) + b1_ref[...]
        z = jnp.maximum(z, 0.0)
        out_ref[...] = (
            jnp.dot(z, w2_ref[...], preferred_element_type=jnp.float32) + b2_ref[...]
        )

    return pl.pallas_call(
        body,
        out_shape=jax.ShapeDtypeStruct((NG, OUT), jnp.float32),
    )


def kernel(x, edge_index, batch, params):
    N, F = x.shape
    E = edge_index.shape[1]
    OUT = params["post_W2"].shape[1]
    src = edge_index[0]
    dst = edge_index[1]
    bn = 5000
    G = N // bn
    batch3 = batch.reshape(G, 1, bn)

    src2d = src.reshape(E // 128, 128)
    dst2d = dst.reshape(E // 128, 128)

    pre = _make_pre(N, F, bn)
    h, p0 = pre(x, batch3, params["pre_W"], params["pre_b"].reshape(1, HID))

    segsum = _make_edge_segsum(N, E)
    mlp_plain = _make_mlp(N, bn, False)
    mlp_res = _make_mlp(N, bn, True)

    pools = [p0]
    hres = h
    for l in range(4):
        agg2 = segsum(h.reshape(2 * N, HALF), src2d, dst2d).reshape(2, N, HALF)
        w1 = params["conv%d_W1" % l]
        b1 = params["conv%d_b1" % l].reshape(1, HID)
        w2 = params["conv%d_W2" % l]
        b2 = params["conv%d_b2" % l].reshape(1, HID)
        if l & 1:
            h, hres, pc = mlp_res(h, agg2, hres, batch3, w1, b1, w2, b2)
        else:
            h, pc = mlp_plain(h, agg2, batch3, w1, b1, w2, b2)
        pools.append(pc)

    pooled = jnp.concatenate(pools, axis=1)
    post = _make_post(OUT, pooled.shape[1])
    return post(
        pooled,
        params["post_W1"],
        params["post_b1"].reshape(1, HID),
        params["post_W2"],
        params["post_b2"].reshape(1, OUT),
    )


# speedup vs baseline: 1.8981x; 1.1029x over previous
import functools

import jax
import jax.numpy as jnp
from jax import lax
from jax.experimental import pallas as pl
from jax.experimental.pallas import tpu as pltpu
from jax.experimental.pallas import tpu_sc as plsc

HID = 64
NG = 64
HALF = HID // 2


@functools.cache
def _make_edge_segsum(N, E):
    NC, NS, CH = 2, 16, 128
    NB = 5
    SLK = 30
    nchunk = E // CH
    full_slabs = (nchunk // SLK) // NS * NS
    slabs_per_tile = full_slabs // NS
    groups_per_slab = SLK // NB
    tail0 = full_slabs * SLK
    tail_chunks = nchunk - tail0
    tail_iter = (tail_chunks + NS - 1) // NS
    ZR = 128
    nz_full = N // ZR
    ztail = N - nz_full * ZR
    nz_iter = (nz_full + NS - 1) // NS

    mesh = plsc.VectorSubcoreMesh(core_axis_name="c", subcore_axis_name="s")

    @functools.partial(
        pl.kernel,
        out_type=jax.ShapeDtypeStruct((NC * N, 4 * HALF), jnp.float32),
        mesh=mesh,
        scratch_types=[
            pltpu.VMEM((SLK, CH), jnp.int32),
            pltpu.VMEM((SLK, CH), jnp.int32),
            pltpu.VMEM((NB, CH), jnp.int32),
            pltpu.VMEM((NB, CH, HALF), jnp.float32),
            pltpu.VMEM_SHARED((N, HALF), jnp.float32),
        ] + [pltpu.SemaphoreType.DMA] * (2 * NB),
        compiler_params=pltpu.CompilerParams(use_tc_tiling_on_sc=False),
    )
    def seg(h2, src2d, dst2d, out, srcsl, dstsl, idxb, rows, acc, *sems):
        gsems = sems[:NB]
        ssems = sems[NB:]
        c = lax.axis_index("c")
        s = lax.axis_index("s")
        zero16 = jnp.zeros((16,), jnp.float32)

        def zb(i, carry):
            for b in range(NB):
                rows[b, i, pl.ds(0, 16)] = zero16
                rows[b, i, pl.ds(16, 16)] = zero16
            return carry

        lax.fori_loop(0, CH, zb, 0)

        def zr(k, carry):
            zc = s + k * NS

            @pl.when(zc < nz_full)
            def _():
                pltpu.sync_copy(rows.at[0], acc.at[pl.ds(zc * ZR, ZR)])

            return carry

        lax.fori_loop(0, nz_iter, zr, 0)
        if ztail:
            @pl.when(s == 0)
            def _():
                pltpu.sync_copy(rows.at[0].at[pl.ds(0, ztail)],
                                acc.at[pl.ds(nz_full * ZR, ztail)])
        plsc.subcore_barrier()

        def slab_body(sl_i, carry):
            slab = s * slabs_per_tile + sl_i
            base_chunk = slab * SLK
            pltpu.sync_copy(src2d.at[pl.ds(base_chunk, SLK)], srcsl)
            pltpu.sync_copy(dst2d.at[pl.ds(base_chunk, SLK)], dstsl)

            def group(gi, gcarry):
                base = gi * NB
                for b in range(NB):
                    j = base + b

                    @pl.when(gi > 0)
                    def _():
                        pltpu.make_async_copy(
                            rows.at[b], acc.at[dstsl.at[0]], ssems[b]
                        ).wait()

                    for i in range(CH // 16):
                        slc = pl.ds(i * 16, 16)
                        idxb[b, slc] = srcsl[j, slc] * 2 + c
                    pltpu.async_copy(h2.at[idxb.at[b]], rows.at[b], gsems[b])
                for b in range(NB):
                    j = base + b
                    pltpu.make_async_copy(
                        h2.at[idxb.at[b]], rows.at[b], gsems[b]
                    ).wait()
                    pltpu.async_copy(
                        rows.at[b], acc.at[dstsl.at[j]], ssems[b], add=True
                    )
                return gcarry

            lax.fori_loop(0, groups_per_slab, group, 0)
            for b in range(NB):
                pltpu.make_async_copy(
                    rows.at[b], acc.at[dstsl.at[0]], ssems[b]
                ).wait()
            return carry

        lax.fori_loop(0, slabs_per_tile, slab_body, 0)

        if tail_chunks:
            def tail(k, carry):
                t = tail0 + s + k * NS

                @pl.when(t < nchunk)
                def _():
                    pltpu.sync_copy(src2d.at[t], srcsl.at[0])
                    pltpu.sync_copy(dst2d.at[t], dstsl.at[0])
                    for i in range(CH // 16):
                        slc = pl.ds(i * 16, 16)
                        idxb[0, slc] = srcsl[0, slc] * 2 + c
                    pltpu.async_copy(
                        h2.at[idxb.at[0]], rows.at[0], gsems[0]
                    ).wait()
                    pltpu.sync_copy(rows.at[0], acc.at[dstsl.at[0]], add=True)

                return carry

            lax.fori_loop(0, tail_iter, tail, 0)

        plsc.subcore_barrier()

        def wb(k, carry):
            zc = s + k * NS

            @pl.when(zc < nz_full)
            def _():
                pltpu.sync_copy(acc.at[pl.ds(zc * ZR, ZR)],
                                out.at[pl.ds(c * N + zc * ZR, ZR),
                                       pl.ds(0, HALF)])

            return carry

        lax.fori_loop(0, nz_iter, wb, 0)
        if ztail:
            @pl.when(s == 0)
            def _():
                pltpu.sync_copy(acc.at[pl.ds(nz_full * ZR, ztail)],
                                out.at[pl.ds(c * N + nz_full * ZR, ztail),
                                       pl.ds(0, HALF)])

    return seg


def _pool_contrib(batch_ref, hn, bn):
    bt = batch_ref[0]
    ohT = (lax.broadcasted_iota(jnp.int32, (NG, bn), 0) == bt).astype(jnp.float32)
    return jnp.dot(ohT, hn, preferred_element_type=jnp.float32)


@functools.cache
def _make_pre(N, F, bn):
    G = N // bn

    def body(x_ref, batch_ref, w_ref, b_ref, h_ref, pool_ref):
        i = pl.program_id(0)
        h = jnp.dot(x_ref[...], w_ref[...], preferred_element_type=jnp.float32)
        h = h + b_ref[...]
        h_ref[...] = h
        contrib = _pool_contrib(batch_ref, h, bn)

        @pl.when(i == 0)
        def _():
            pool_ref[...] = contrib

        @pl.when(i != 0)
        def _():
            pool_ref[...] = pool_ref[...] + contrib

    return pl.pallas_call(
        body,
        grid=(G,),
        in_specs=[
            pl.BlockSpec((bn, F), lambda i: (i, 0)),
            pl.BlockSpec((1, 1, bn), lambda i: (i, 0, 0)),
            pl.BlockSpec((F, HID), lambda i: (0, 0)),
            pl.BlockSpec((1, HID), lambda i: (0, 0)),
        ],
        out_specs=[
            pl.BlockSpec((bn, HID), lambda i: (i, 0)),
            pl.BlockSpec((NG, HID), lambda i: (0, 0)),
        ],
        out_shape=[
            jax.ShapeDtypeStruct((N, HID), jnp.float32),
            jax.ShapeDtypeStruct((NG, HID), jnp.float32),
        ],
    )


@functools.cache
def _make_mlp(N, bn, residual):
    G = N // bn

    def body(*refs):
        if residual:
            (h_ref, agg_ref, hres_ref, batch_ref, w1_ref, b1_ref, w2_ref,
             b2_ref, hout_ref, hresout_ref, pool_ref) = refs
        else:
            (h_ref, agg_ref, batch_ref, w1_ref, b1_ref, w2_ref, b2_ref,
             hout_ref, pool_ref) = refs
        i = pl.program_id(0)
        a = jnp.concatenate(
            [agg_ref[0, :, :HALF], agg_ref[1, :, :HALF]], axis=1)
        z = h_ref[...] + a
        z = jnp.dot(z, w1_ref[...], preferred_element_type=jnp.float32) + b1_ref[...]
        z = jnp.maximum(z, 0.0)
        z = jnp.dot(z, w2_ref[...], preferred_element_type=jnp.float32) + b2_ref[...]
        if residual:
            z = z + hres_ref[...]
            hresout_ref[...] = z
        hn = jnp.maximum(z, 0.0)
        hout_ref[...] = hn
        contrib = _pool_contrib(batch_ref, hn, bn)

        @pl.when(i == 0)
        def _():
            pool_ref[...] = contrib

        @pl.when(i != 0)
        def _():
            pool_ref[...] = pool_ref[...] + contrib

    in_specs = [
        pl.BlockSpec((bn, HID), lambda i: (i, 0)),
        pl.BlockSpec((2, bn, 4 * HALF), lambda i: (0, i, 0)),
    ]
    if residual:
        in_specs.append(pl.BlockSpec((bn, HID), lambda i: (i, 0)))
    in_specs += [
        pl.BlockSpec((1, 1, bn), lambda i: (i, 0, 0)),
        pl.BlockSpec((HID, HID), lambda i: (0, 0)),
        pl.BlockSpec((1, HID), lambda i: (0, 0)),
        pl.BlockSpec((HID, HID), lambda i: (0, 0)),
        pl.BlockSpec((1, HID), lambda i: (0, 0)),
    ]
    out_specs = [pl.BlockSpec((bn, HID), lambda i: (i, 0))]
    out_shape = [jax.ShapeDtypeStruct((N, HID), jnp.float32)]
    if residual:
        out_specs.append(pl.BlockSpec((bn, HID), lambda i: (i, 0)))
        out_shape.append(jax.ShapeDtypeStruct((N, HID), jnp.float32))
    out_specs.append(pl.BlockSpec((NG, HID), lambda i: (0, 0)))
    out_shape.append(jax.ShapeDtypeStruct((NG, HID), jnp.float32))

    return pl.pallas_call(
        body,
        grid=(G,),
        in_specs=in_specs,
        out_specs=out_specs,
        out_shape=out_shape,
        compiler_params=pltpu.CompilerParams(
            allow_input_fusion=[True] * len(in_specs)),
    )


@functools.cache
def _make_post(OUT, CAT):
    def body(p_ref, w1_ref, b1_ref, w2_ref, b2_ref, out_ref):
        p = p_ref[...]
        z = jnp.dot(p, w1_ref[...], preferred_element_type=jnp.float32) + b1_ref[...]
        z = jnp.maximum(z, 0.0)
        out_ref[...] = (
            jnp.dot(z, w2_ref[...], preferred_element_type=jnp.float32) + b2_ref[...]
        )

    return pl.pallas_call(
        body,
        out_shape=jax.ShapeDtypeStruct((NG, OUT), jnp.float32),
    )


def kernel(x, edge_index, batch, params):
    N, F = x.shape
    E = edge_index.shape[1]
    OUT = params["post_W2"].shape[1]
    src = edge_index[0]
    dst = edge_index[1]
    bn = 5000
    G = N // bn
    batch3 = batch.reshape(G, 1, bn)

    src2d = src.reshape(E // 128, 128)
    dst2d = dst.reshape(E // 128, 128)

    pre = _make_pre(N, F, bn)
    h, p0 = pre(x, batch3, params["pre_W"], params["pre_b"].reshape(1, HID))

    segsum = _make_edge_segsum(N, E)
    mlp_plain = _make_mlp(N, bn, False)
    mlp_res = _make_mlp(N, bn, True)

    pools = [p0]
    hres = h
    for l in range(4):
        agg2 = segsum(h.reshape(2 * N, HALF), src2d, dst2d).reshape(2, N, 4 * HALF)
        w1 = params["conv%d_W1" % l]
        b1 = params["conv%d_b1" % l].reshape(1, HID)
        w2 = params["conv%d_W2" % l]
        b2 = params["conv%d_b2" % l].reshape(1, HID)
        if l & 1:
            h, hres, pc = mlp_res(h, agg2, hres, batch3, w1, b1, w2, b2)
        else:
            h, pc = mlp_plain(h, agg2, batch3, w1, b1, w2, b2)
        pools.append(pc)

    pooled = jnp.concatenate(pools, axis=1)
    post = _make_post(OUT, pooled.shape[1])
    return post(
        pooled,
        params["post_W1"],
        params["post_b1"].reshape(1, HID),
        params["post_W2"],
        params["post_b2"].reshape(1, OUT),
    )


# speedup vs baseline: 1.9699x; 1.0378x over previous
import functools

import jax
import jax.numpy as jnp
from jax import lax
from jax.experimental import pallas as pl
from jax.experimental.pallas import tpu as pltpu
from jax.experimental.pallas import tpu_sc as plsc

HID = 64
NG = 64
HALF = HID // 2


@functools.cache
def _make_edge_segsum(N, E):
    NC, NS, CH = 2, 16, 128
    NB = 5
    SLK = 30
    nchunk = E // CH
    full_slabs = (nchunk // SLK) // NS * NS
    slabs_per_tile = full_slabs // NS
    groups_per_slab = SLK // NB
    tail0 = full_slabs * SLK
    tail_chunks = nchunk - tail0
    tail_iter = (tail_chunks + NS - 1) // NS
    ZR = 128
    nz_full = N // ZR
    ztail = N - nz_full * ZR
    nz_iter = (nz_full + NS - 1) // NS

    mesh = plsc.VectorSubcoreMesh(core_axis_name="c", subcore_axis_name="s")

    @functools.partial(
        pl.kernel,
        out_type=jax.ShapeDtypeStruct((NC * N, 4 * HALF), jnp.float32),
        mesh=mesh,
        scratch_types=[
            pltpu.VMEM((SLK, CH), jnp.int32),
            pltpu.VMEM((SLK, CH), jnp.int32),
            pltpu.VMEM((NB, CH), jnp.int32),
            pltpu.VMEM((NB, CH, HALF), jnp.float32),
            pltpu.VMEM_SHARED((N, HALF), jnp.float32),
        ] + [pltpu.SemaphoreType.DMA] * (2 * NB + 1),
        compiler_params=pltpu.CompilerParams(use_tc_tiling_on_sc=False),
    )
    def seg(h2, src2d, dst2d, out, srcsl, dstsl, idxb, rows, acc, *sems):
        gsems = sems[:NB]
        ssems = sems[NB:2 * NB]
        slsem = sems[2 * NB]
        c = lax.axis_index("c")
        s = lax.axis_index("s")
        zero16 = jnp.zeros((16,), jnp.float32)

        def zb(i, carry):
            for b in range(NB):
                rows[b, i, pl.ds(0, 16)] = zero16
                rows[b, i, pl.ds(16, 16)] = zero16
            return carry

        lax.fori_loop(0, CH, zb, 0)

        def zr(k, carry):
            zc = s + k * NS

            @pl.when(zc < nz_full)
            def _():
                pltpu.sync_copy(rows.at[0], acc.at[pl.ds(zc * ZR, ZR)])

            return carry

        lax.fori_loop(0, nz_iter, zr, 0)
        if ztail:
            @pl.when(s == 0)
            def _():
                pltpu.sync_copy(rows.at[0].at[pl.ds(0, ztail)],
                                acc.at[pl.ds(nz_full * ZR, ztail)])
        plsc.subcore_barrier()

        def slab_body(sl_i, carry):
            slab = s * slabs_per_tile + sl_i
            base_chunk = slab * SLK
            pltpu.async_copy(src2d.at[pl.ds(base_chunk, SLK)], srcsl, slsem)
            pltpu.async_copy(dst2d.at[pl.ds(base_chunk, SLK)], dstsl, slsem)
            pltpu.make_async_copy(
                src2d.at[pl.ds(base_chunk, SLK)], srcsl, slsem).wait()
            pltpu.make_async_copy(
                dst2d.at[pl.ds(base_chunk, SLK)], dstsl, slsem).wait()

            def group(gi, gcarry):
                base = gi * NB
                for b in range(NB):
                    j = base + b

                    @pl.when(gi > 0)
                    def _():
                        pltpu.make_async_copy(
                            rows.at[b], acc.at[dstsl.at[0]], ssems[b]
                        ).wait()

                    for i in range(CH // 16):
                        slc = pl.ds(i * 16, 16)
                        idxb[b, slc] = srcsl[j, slc] * 2 + c
                    pltpu.async_copy(h2.at[idxb.at[b]], rows.at[b], gsems[b])
                for b in range(NB):
                    j = base + b
                    pltpu.make_async_copy(
                        h2.at[idxb.at[b]], rows.at[b], gsems[b]
                    ).wait()
                    pltpu.async_copy(
                        rows.at[b], acc.at[dstsl.at[j]], ssems[b], add=True
                    )
                return gcarry

            lax.fori_loop(0, groups_per_slab, group, 0)
            for b in range(NB):
                pltpu.make_async_copy(
                    rows.at[b], acc.at[dstsl.at[0]], ssems[b]
                ).wait()
            return carry

        lax.fori_loop(0, slabs_per_tile, slab_body, 0)

        if tail_chunks:
            def tail(k, carry):
                t = tail0 + s + k * NS

                @pl.when(t < nchunk)
                def _():
                    pltpu.sync_copy(src2d.at[t], srcsl.at[0])
                    pltpu.sync_copy(dst2d.at[t], dstsl.at[0])
                    for i in range(CH // 16):
                        slc = pl.ds(i * 16, 16)
                        idxb[0, slc] = srcsl[0, slc] * 2 + c
                    pltpu.async_copy(
                        h2.at[idxb.at[0]], rows.at[0], gsems[0]
                    ).wait()
                    pltpu.sync_copy(rows.at[0], acc.at[dstsl.at[0]], add=True)

                return carry

            lax.fori_loop(0, tail_iter, tail, 0)

        plsc.subcore_barrier()

        def wb(k, carry):
            zc = s + k * NS

            @pl.when(zc < nz_full)
            def _():
                pltpu.sync_copy(acc.at[pl.ds(zc * ZR, ZR)],
                                out.at[pl.ds(c * N + zc * ZR, ZR),
                                       pl.ds(0, HALF)])

            return carry

        lax.fori_loop(0, nz_iter, wb, 0)
        if ztail:
            @pl.when(s == 0)
            def _():
                pltpu.sync_copy(acc.at[pl.ds(nz_full * ZR, ztail)],
                                out.at[pl.ds(c * N + nz_full * ZR, ztail),
                                       pl.ds(0, HALF)])

    return seg


def _pool_contrib(batch_ref, hn, bn):
    bt = batch_ref[0]
    ohT = (lax.broadcasted_iota(jnp.int32, (NG, bn), 0) == bt).astype(jnp.float32)
    return jnp.dot(ohT, hn, preferred_element_type=jnp.float32)


@functools.cache
def _make_pre(N, F, bn):
    G = N // bn

    def body(x_ref, batch_ref, w_ref, b_ref, h_ref, pool_ref):
        i = pl.program_id(0)
        h = jnp.dot(x_ref[...], w_ref[...], preferred_element_type=jnp.float32)
        h = h + b_ref[...]
        h_ref[...] = h
        contrib = _pool_contrib(batch_ref, h, bn)

        @pl.when(i == 0)
        def _():
            pool_ref[...] = contrib

        @pl.when(i != 0)
        def _():
            pool_ref[...] = pool_ref[...] + contrib

    return pl.pallas_call(
        body,
        grid=(G,),
        in_specs=[
            pl.BlockSpec((bn, F), lambda i: (i, 0)),
            pl.BlockSpec((1, 1, bn), lambda i: (i, 0, 0)),
            pl.BlockSpec((F, HID), lambda i: (0, 0)),
            pl.BlockSpec((1, HID), lambda i: (0, 0)),
        ],
        out_specs=[
            pl.BlockSpec((bn, HID), lambda i: (i, 0)),
            pl.BlockSpec((NG, HID), lambda i: (0, 0)),
        ],
        out_shape=[
            jax.ShapeDtypeStruct((N, HID), jnp.float32),
            jax.ShapeDtypeStruct((NG, HID), jnp.float32),
        ],
    )


@functools.cache
def _make_mlp(N, bn, residual):
    G = N // bn

    def body(*refs):
        if residual:
            (h_ref, agg_ref, hres_ref, batch_ref, w1_ref, b1_ref, w2_ref,
             b2_ref, hout_ref, hresout_ref, pool_ref) = refs
        else:
            (h_ref, agg_ref, batch_ref, w1_ref, b1_ref, w2_ref, b2_ref,
             hout_ref, pool_ref) = refs
        i = pl.program_id(0)
        a = jnp.concatenate(
            [agg_ref[0, :, :HALF], agg_ref[1, :, :HALF]], axis=1)
        z = h_ref[...] + a
        z = jnp.dot(z, w1_ref[...], preferred_element_type=jnp.float32) + b1_ref[...]
        z = jnp.maximum(z, 0.0)
        z = jnp.dot(z, w2_ref[...], preferred_element_type=jnp.float32) + b2_ref[...]
        if residual:
            z = z + hres_ref[...]
            hresout_ref[...] = z
        hn = jnp.maximum(z, 0.0)
        hout_ref[...] = hn
        contrib = _pool_contrib(batch_ref, hn, bn)

        @pl.when(i == 0)
        def _():
            pool_ref[...] = contrib

        @pl.when(i != 0)
        def _():
            pool_ref[...] = pool_ref[...] + contrib

    in_specs = [
        pl.BlockSpec((bn, HID), lambda i: (i, 0)),
        pl.BlockSpec((2, bn, 4 * HALF), lambda i: (0, i, 0)),
    ]
    if residual:
        in_specs.append(pl.BlockSpec((bn, HID), lambda i: (i, 0)))
    in_specs += [
        pl.BlockSpec((1, 1, bn), lambda i: (i, 0, 0)),
        pl.BlockSpec((HID, HID), lambda i: (0, 0)),
        pl.BlockSpec((1, HID), lambda i: (0, 0)),
        pl.BlockSpec((HID, HID), lambda i: (0, 0)),
        pl.BlockSpec((1, HID), lambda i: (0, 0)),
    ]
    out_specs = [pl.BlockSpec((bn, HID), lambda i: (i, 0))]
    out_shape = [jax.ShapeDtypeStruct((N, HID), jnp.float32)]
    if residual:
        out_specs.append(pl.BlockSpec((bn, HID), lambda i: (i, 0)))
        out_shape.append(jax.ShapeDtypeStruct((N, HID), jnp.float32))
    out_specs.append(pl.BlockSpec((NG, HID), lambda i: (0, 0)))
    out_shape.append(jax.ShapeDtypeStruct((NG, HID), jnp.float32))

    return pl.pallas_call(
        body,
        grid=(G,),
        in_specs=in_specs,
        out_specs=out_specs,
        out_shape=out_shape,
        compiler_params=pltpu.CompilerParams(
            allow_input_fusion=[True] * len(in_specs)),
    )


@functools.cache
def _make_post(OUT, CAT):
    def body(p_ref, w1_ref, b1_ref, w2_ref, b2_ref, out_ref):
        p = p_ref[...]
        z = jnp.dot(p, w1_ref[...], preferred_element_type=jnp.float32) + b1_ref[...]
        z = jnp.maximum(z, 0.0)
        out_ref[...] = (
            jnp.dot(z, w2_ref[...], preferred_element_type=jnp.float32) + b2_ref[...]
        )

    return pl.pallas_call(
        body,
        out_shape=jax.ShapeDtypeStruct((NG, OUT), jnp.float32),
    )


def kernel(x, edge_index, batch, params):
    N, F = x.shape
    E = edge_index.shape[1]
    OUT = params["post_W2"].shape[1]
    src = edge_index[0]
    dst = edge_index[1]
    bn = 5000
    G = N // bn
    batch3 = batch.reshape(G, 1, bn)

    src2d = src.reshape(E // 128, 128)
    dst2d = dst.reshape(E // 128, 128)

    pre = _make_pre(N, F, bn)
    h, p0 = pre(x, batch3, params["pre_W"], params["pre_b"].reshape(1, HID))

    segsum = _make_edge_segsum(N, E)
    mlp_plain = _make_mlp(N, bn, False)
    mlp_res = _make_mlp(N, bn, True)

    pools = [p0]
    hres = h
    for l in range(4):
        agg2 = segsum(h.reshape(2 * N, HALF), src2d, dst2d).reshape(2, N, 4 * HALF)
        w1 = params["conv%d_W1" % l]
        b1 = params["conv%d_b1" % l].reshape(1, HID)
        w2 = params["conv%d_W2" % l]
        b2 = params["conv%d_b2" % l].reshape(1, HID)
        if l & 1:
            h, hres, pc = mlp_res(h, agg2, hres, batch3, w1, b1, w2, b2)
        else:
            h, pc = mlp_plain(h, agg2, batch3, w1, b1, w2, b2)
        pools.append(pc)

    pooled = jnp.concatenate(pools, axis=1)
    post = _make_post(OUT, pooled.shape[1])
    return post(
        pooled,
        params["post_W1"],
        params["post_b1"].reshape(1, HID),
        params["post_W2"],
        params["post_b2"].reshape(1, OUT),
    )
